# Initial kernel scaffold; baseline (speedup 1.0000x reference)
import jax
import jax.numpy as jnp
from jax.experimental import pallas as pl


def kernel(x, edge_index, batch, W1, b1, Wm1, bm1, Wm2, bm2, W2, b2):
    raise NotImplementedError("write your pallas kernel here")


# speedup vs baseline: 6.1287x; 6.1287x over previous
import functools

import jax
import jax.numpy as jnp
from jax import lax
from jax.experimental import pallas as pl
from jax.experimental.pallas import tpu as pltpu
from jax.experimental.pallas import tpu_sc as plsc

_N = 10000
_E = 320000
_D = 128
_G = 64

_NC = 2
_NS = 16
_NW = _NC * _NS
_CHUNK = 128
_CPW = 80
_EPAD = _NW * _CPW * _CHUNK
_ACC_ROWS = 10240
_ZROWS = _ACC_ROWS // _NS

_R = 1000

_f32 = jnp.float32
_mesh = functools.partial(
    plsc.VectorSubcoreMesh, core_axis_name="c", subcore_axis_name="s")


def _sc_degree_body(dst_hbm, out_hbm, dst_v, ones_v, zb_v, acc):
    c = lax.axis_index("c")
    s = lax.axis_index("s")
    w = c * _NS + s
    pltpu.sync_copy(dst_hbm.at[w], dst_v)

    @pl.loop(0, _CHUNK)
    def _fill(r):
        ones_v[r, :] = jnp.full((16,), 1.0, _f32)
        zb_v[r, :] = jnp.zeros((16,), _f32)

    @pl.loop(0, _ZROWS // _CHUNK)
    def _zero(k):
        pltpu.sync_copy(zb_v, acc.at[pl.ds(s * _ZROWS + k * _CHUNK, _CHUNK)])

    plsc.subcore_barrier()

    @pl.loop(0, _CPW)
    def _scatter(j):
        pltpu.sync_copy(ones_v, acc.at[dst_v.at[j]], add=True)

    plsc.subcore_barrier()
    pltpu.sync_copy(acc.at[pl.ds(s * _ZROWS, _ZROWS)],
                    out_hbm.at[c, pl.ds(s * _ZROWS, _ZROWS)])


def _sc_degree(dst3):
    return pl.kernel(
        _sc_degree_body,
        out_type=jax.ShapeDtypeStruct((_NC, _ACC_ROWS, 16), _f32),
        mesh=_mesh(),
        scratch_types=[
            pltpu.VMEM((_CPW, _CHUNK), jnp.int32),
            pltpu.VMEM((_CHUNK, 16), _f32),
            pltpu.VMEM((_CHUNK, 16), _f32),
            pltpu.VMEM_SHARED((_ACC_ROWS, 16), _f32),
        ],
    )(dst3)


def _sc_scatter_body(h_hbm, src_hbm, dst_hbm, out_hbm,
                     src_v, dst_v, rows_v, acc, sem):
    c = lax.axis_index("c")
    s = lax.axis_index("s")
    w = c * _NS + s
    pltpu.sync_copy(src_hbm.at[w], src_v)
    pltpu.sync_copy(dst_hbm.at[w], dst_v)

    @pl.loop(0, _CHUNK)
    def _fill(r):
        for cc in range(_D // 16):
            rows_v[r, pl.ds(cc * 16, 16)] = jnp.zeros((16,), _f32)

    @pl.loop(0, _ZROWS // _CHUNK)
    def _zero(k):
        pltpu.sync_copy(rows_v, acc.at[pl.ds(s * _ZROWS + k * _CHUNK, _CHUNK)])

    plsc.subcore_barrier()

    @pl.loop(0, _CPW)
    def _edges(j):
        pltpu.async_copy(h_hbm.at[src_v.at[j]], rows_v, sem).wait()
        pltpu.sync_copy(rows_v, acc.at[dst_v.at[j]], add=True)

    plsc.subcore_barrier()
    pltpu.sync_copy(acc.at[pl.ds(s * _ZROWS, _ZROWS)],
                    out_hbm.at[c, pl.ds(s * _ZROWS, _ZROWS)])


def _sc_scatter(h, src3, dst3):
    return pl.kernel(
        _sc_scatter_body,
        out_type=jax.ShapeDtypeStruct((_NC, _ACC_ROWS, _D), _f32),
        mesh=_mesh(),
        scratch_types=[
            pltpu.VMEM((_CPW, _CHUNK), jnp.int32),
            pltpu.VMEM((_CPW, _CHUNK), jnp.int32),
            pltpu.VMEM((_CHUNK, _D), _f32),
            pltpu.VMEM_SHARED((_ACC_ROWS, _D), _f32),
            pltpu.SemaphoreType.DMA,
        ],
    )(h, src3, dst3)


def _dot(a, b):
    return lax.dot_general(a, b, (((1,), (0,)), ((), ())),
                           preferred_element_type=_f32,
                           precision=lax.Precision.HIGHEST)


def _tc_prep_body(dacc_ref, x_ref, w_ref, dinv_ref, h_ref):
    deg = dacc_ref[0, :, 0:1] + dacc_ref[1, :, 0:1] + 1.0
    dinv = jnp.broadcast_to(lax.rsqrt(deg), (_R, _D))
    dinv_ref[...] = dinv
    h_ref[...] = _dot(x_ref[...], w_ref[...]) * dinv


def _tc_prep(dacc, x, w1):
    return pl.pallas_call(
        _tc_prep_body,
        grid=(_N // _R,),
        in_specs=[
            pl.BlockSpec((_NC, _R, 16), lambda i: (0, i, 0)),
            pl.BlockSpec((_R, _D), lambda i: (i, 0)),
            pl.BlockSpec((_D, _D), lambda i: (0, 0)),
        ],
        out_specs=[
            pl.BlockSpec((_R, _D), lambda i: (i, 0)),
            pl.BlockSpec((_R, _D), lambda i: (i, 0)),
        ],
        out_shape=[
            jax.ShapeDtypeStruct((_N, _D), _f32),
            jax.ShapeDtypeStruct((_N, _D), _f32),
        ],
    )(dacc, x, w1)


def _tc_mid_body(acc_ref, hprev_ref, dinv_ref, b_ref, w_ref, h_ref):
    dinv = dinv_ref[...]
    ssum = acc_ref[0] + acc_ref[1] + hprev_ref[...]
    xl = jnp.maximum(dinv * ssum + b_ref[...], 0.0)
    h_ref[...] = _dot(xl, w_ref[...]) * dinv


def _tc_mid(acc, hprev, dinv, b, w):
    return pl.pallas_call(
        _tc_mid_body,
        grid=(_N // _R,),
        in_specs=[
            pl.BlockSpec((_NC, _R, _D), lambda i: (0, i, 0)),
            pl.BlockSpec((_R, _D), lambda i: (i, 0)),
            pl.BlockSpec((_R, _D), lambda i: (i, 0)),
            pl.BlockSpec((1, _D), lambda i: (0, 0)),
            pl.BlockSpec((_D, _D), lambda i: (0, 0)),
        ],
        out_specs=pl.BlockSpec((_R, _D), lambda i: (i, 0)),
        out_shape=jax.ShapeDtypeStruct((_N, _D), _f32),
    )(acc, hprev, dinv, b.reshape(1, _D), w)


def _tc_final_body(acc_ref, h4_ref, dinv_ref, b_ref, batch_ref, out_ref,
                   sum_acc, cnt_acc):
    i = pl.program_id(0)
    h = dinv_ref[...] * (acc_ref[0] + acc_ref[1] + h4_ref[...]) + b_ref[...]
    bt = batch_ref[0, 0, :]
    gids = lax.broadcasted_iota(jnp.int32, (_G, _R), 0)
    onehot = (gids == bt[None, :]).astype(_f32)
    psum = _dot(onehot, h)
    cnt = jnp.broadcast_to(jnp.sum(onehot, axis=1, keepdims=True), (_G, _D))

    @pl.when(i == 0)
    def _init():
        sum_acc[...] = psum
        cnt_acc[...] = cnt

    @pl.when(i > 0)
    def _accum():
        sum_acc[...] += psum
        cnt_acc[...] += cnt

    @pl.when(i == _N // _R - 1)
    def _finalize():
        pool = sum_acc[...] / jnp.maximum(cnt_acc[...], 1.0)
        nrm = jnp.sqrt(jnp.sum(pool * pool, axis=1, keepdims=True))
        out_ref[...] = pool / jnp.maximum(nrm, 1e-12)


def _tc_final(acc, h4, dinv, b2, batch3):
    return pl.pallas_call(
        _tc_final_body,
        grid=(_N // _R,),
        in_specs=[
            pl.BlockSpec((_NC, _R, _D), lambda i: (0, i, 0)),
            pl.BlockSpec((_R, _D), lambda i: (i, 0)),
            pl.BlockSpec((_R, _D), lambda i: (i, 0)),
            pl.BlockSpec((1, _D), lambda i: (0, 0)),
            pl.BlockSpec((1, 1, _R), lambda i: (i, 0, 0)),
        ],
        out_specs=pl.BlockSpec((_G, _D), lambda i: (0, 0)),
        out_shape=jax.ShapeDtypeStruct((_G, _D), _f32),
        scratch_shapes=[
            pltpu.VMEM((_G, _D), _f32),
            pltpu.VMEM((_G, _D), _f32),
        ],
    )(acc, h4, dinv, b2.reshape(1, _D), batch3)


def kernel(x, edge_index, batch, W1, b1, Wm1, bm1, Wm2, bm2, W2, b2):
    pad = _EPAD - _E
    src3 = jnp.concatenate(
        [edge_index[0], jnp.zeros((pad,), jnp.int32)]).reshape(_NW, _CPW, _CHUNK)
    dst3 = jnp.concatenate(
        [edge_index[1], jnp.full((pad,), _N, jnp.int32)]).reshape(_NW, _CPW, _CHUNK)
    batch3 = batch.reshape(_N // _R, 1, _R)

    dacc = _sc_degree(dst3)
    dinv, h1 = _tc_prep(dacc, x, W1)
    acc1 = _sc_scatter(h1, src3, dst3)
    h2 = _tc_mid(acc1, h1, dinv, b1, Wm1)
    acc2 = _sc_scatter(h2, src3, dst3)
    h3 = _tc_mid(acc2, h2, dinv, bm1, Wm2)
    acc3 = _sc_scatter(h3, src3, dst3)
    h4 = _tc_mid(acc3, h3, dinv, bm2, W2)
    acc4 = _sc_scatter(h4, src3, dst3)
    return _tc_final(acc4, h4, dinv, b2, batch3)


# speedup vs baseline: 6.7991x; 1.1094x over previous
import functools

import jax
import jax.numpy as jnp
from jax import lax
from jax.experimental import pallas as pl
from jax.experimental.pallas import tpu as pltpu
from jax.experimental.pallas import tpu_sc as plsc

_N = 10000
_E = 320000
_D = 128
_G = 64

_NC = 2
_NS = 16
_NW = _NC * _NS
_CHUNK = 128
_CPW = 80
_EPAD = _NW * _CPW * _CHUNK
_ACC_ROWS = 10240
_ZROWS = _ACC_ROWS // _NS

_R = 1000

_f32 = jnp.float32
_mesh = functools.partial(
    plsc.VectorSubcoreMesh, core_axis_name="c", subcore_axis_name="s")


def _sc_degree_body(dst_hbm, out_hbm, dst_v, ones_v, zb_v, acc):
    c = lax.axis_index("c")
    s = lax.axis_index("s")
    w = c * _NS + s
    pltpu.sync_copy(dst_hbm.at[w], dst_v)

    @pl.loop(0, _CHUNK)
    def _fill(r):
        ones_v[r, :] = jnp.full((16,), 1.0, _f32)
        zb_v[r, :] = jnp.zeros((16,), _f32)

    @pl.loop(0, _ZROWS // _CHUNK)
    def _zero(k):
        pltpu.sync_copy(zb_v, acc.at[pl.ds(s * _ZROWS + k * _CHUNK, _CHUNK)])

    plsc.subcore_barrier()

    @pl.loop(0, _CPW)
    def _scatter(j):
        pltpu.sync_copy(ones_v, acc.at[dst_v.at[j]], add=True)

    plsc.subcore_barrier()
    pltpu.sync_copy(acc.at[pl.ds(s * _ZROWS, _ZROWS)],
                    out_hbm.at[c, pl.ds(s * _ZROWS, _ZROWS)])


def _sc_degree(dst3):
    return pl.kernel(
        _sc_degree_body,
        out_type=jax.ShapeDtypeStruct((_NC, _ACC_ROWS, 16), _f32),
        mesh=_mesh(),
        scratch_types=[
            pltpu.VMEM((_CPW, _CHUNK), jnp.int32),
            pltpu.VMEM((_CHUNK, 16), _f32),
            pltpu.VMEM((_CHUNK, 16), _f32),
            pltpu.VMEM_SHARED((_ACC_ROWS, 16), _f32),
        ],
    )(dst3)


_HCPW = _CPW // 2


def _sc_scatter_body(h_hbm, src_hbm, dst_hbm, out_hbm,
                     src_v, dst_v, rows0_v, rows1_v, acc, sem0, sem1):
    c = lax.axis_index("c")
    s = lax.axis_index("s")
    w = c * _NS + s
    rows = (rows0_v, rows1_v)
    sems = (sem0, sem1)

    @pl.loop(0, _CHUNK)
    def _fill(r):
        for cc in range(_D // 16):
            rows0_v[r, pl.ds(cc * 16, 16)] = jnp.zeros((16,), _f32)

    @pl.loop(0, _ZROWS // _CHUNK)
    def _zero(k):
        pltpu.sync_copy(rows0_v, acc.at[pl.ds(s * _ZROWS + k * _CHUNK, _CHUNK)])

    plsc.subcore_barrier()

    for h_half in range(2):
        pltpu.sync_copy(src_hbm.at[w, pl.ds(h_half * _HCPW, _HCPW)], src_v)
        pltpu.sync_copy(dst_hbm.at[w, pl.ds(h_half * _HCPW, _HCPW)], dst_v)
        pltpu.async_copy(h_hbm.at[src_v.at[0]], rows0_v, sem0)
        pltpu.async_copy(h_hbm.at[src_v.at[1]], rows1_v, sem1)

        @pl.loop(0, _HCPW // 2 - 1)
        def _edges(j2):
            for p in range(2):
                j = 2 * j2 + p
                pltpu.make_async_copy(h_hbm.at[src_v.at[j]],
                                      rows[p], sems[p]).wait()
                pltpu.sync_copy(rows[p], acc.at[dst_v.at[j]], add=True)
                pltpu.async_copy(h_hbm.at[src_v.at[j + 2]], rows[p], sems[p])

        for p in range(2):
            j = _HCPW - 2 + p
            pltpu.make_async_copy(h_hbm.at[src_v.at[j]],
                                  rows[p], sems[p]).wait()
            pltpu.sync_copy(rows[p], acc.at[dst_v.at[j]], add=True)

    plsc.subcore_barrier()
    pltpu.sync_copy(acc.at[pl.ds(s * _ZROWS, _ZROWS)],
                    out_hbm.at[c, pl.ds(s * _ZROWS, _ZROWS)])


def _sc_scatter(h, src3, dst3):
    return pl.kernel(
        _sc_scatter_body,
        out_type=jax.ShapeDtypeStruct((_NC, _ACC_ROWS, _D), _f32),
        mesh=_mesh(),
        scratch_types=[
            pltpu.VMEM((_HCPW, _CHUNK), jnp.int32),
            pltpu.VMEM((_HCPW, _CHUNK), jnp.int32),
            pltpu.VMEM((_CHUNK, _D), _f32),
            pltpu.VMEM((_CHUNK, _D), _f32),
            pltpu.VMEM_SHARED((_ACC_ROWS, _D), _f32),
            pltpu.SemaphoreType.DMA,
            pltpu.SemaphoreType.DMA,
        ],
    )(h, src3, dst3)


def _dot(a, b):
    return lax.dot_general(a, b, (((1,), (0,)), ((), ())),
                           preferred_element_type=_f32,
                           precision=lax.Precision.HIGHEST)


def _tc_prep_body(dacc_ref, x_ref, w_ref, dinv_ref, h_ref):
    deg = dacc_ref[0, :, 0:1] + dacc_ref[1, :, 0:1] + 1.0
    dinv = jnp.broadcast_to(lax.rsqrt(deg), (_R, _D))
    dinv_ref[...] = dinv
    h_ref[...] = _dot(x_ref[...], w_ref[...]) * dinv


def _tc_prep(dacc, x, w1):
    return pl.pallas_call(
        _tc_prep_body,
        grid=(_N // _R,),
        in_specs=[
            pl.BlockSpec((_NC, _R, 16), lambda i: (0, i, 0)),
            pl.BlockSpec((_R, _D), lambda i: (i, 0)),
            pl.BlockSpec((_D, _D), lambda i: (0, 0)),
        ],
        out_specs=[
            pl.BlockSpec((_R, _D), lambda i: (i, 0)),
            pl.BlockSpec((_R, _D), lambda i: (i, 0)),
        ],
        out_shape=[
            jax.ShapeDtypeStruct((_N, _D), _f32),
            jax.ShapeDtypeStruct((_N, _D), _f32),
        ],
    )(dacc, x, w1)


def _tc_mid_body(acc_ref, hprev_ref, dinv_ref, b_ref, w_ref, h_ref):
    dinv = dinv_ref[...]
    ssum = acc_ref[0] + acc_ref[1] + hprev_ref[...]
    xl = jnp.maximum(dinv * ssum + b_ref[...], 0.0)
    h_ref[...] = _dot(xl, w_ref[...]) * dinv


def _tc_mid(acc, hprev, dinv, b, w):
    return pl.pallas_call(
        _tc_mid_body,
        grid=(_N // _R,),
        in_specs=[
            pl.BlockSpec((_NC, _R, _D), lambda i: (0, i, 0)),
            pl.BlockSpec((_R, _D), lambda i: (i, 0)),
            pl.BlockSpec((_R, _D), lambda i: (i, 0)),
            pl.BlockSpec((1, _D), lambda i: (0, 0)),
            pl.BlockSpec((_D, _D), lambda i: (0, 0)),
        ],
        out_specs=pl.BlockSpec((_R, _D), lambda i: (i, 0)),
        out_shape=jax.ShapeDtypeStruct((_N, _D), _f32),
    )(acc, hprev, dinv, b.reshape(1, _D), w)


def _tc_final_body(acc_ref, h4_ref, dinv_ref, b_ref, batch_ref, out_ref,
                   sum_acc, cnt_acc):
    i = pl.program_id(0)
    h = dinv_ref[...] * (acc_ref[0] + acc_ref[1] + h4_ref[...]) + b_ref[...]
    bt = batch_ref[0, 0, :]
    gids = lax.broadcasted_iota(jnp.int32, (_G, _R), 0)
    onehot = (gids == bt[None, :]).astype(_f32)
    psum = _dot(onehot, h)
    cnt = jnp.broadcast_to(jnp.sum(onehot, axis=1, keepdims=True), (_G, _D))

    @pl.when(i == 0)
    def _init():
        sum_acc[...] = psum
        cnt_acc[...] = cnt

    @pl.when(i > 0)
    def _accum():
        sum_acc[...] += psum
        cnt_acc[...] += cnt

    @pl.when(i == _N // _R - 1)
    def _finalize():
        pool = sum_acc[...] / jnp.maximum(cnt_acc[...], 1.0)
        nrm = jnp.sqrt(jnp.sum(pool * pool, axis=1, keepdims=True))
        out_ref[...] = pool / jnp.maximum(nrm, 1e-12)


def _tc_final(acc, h4, dinv, b2, batch3):
    return pl.pallas_call(
        _tc_final_body,
        grid=(_N // _R,),
        in_specs=[
            pl.BlockSpec((_NC, _R, _D), lambda i: (0, i, 0)),
            pl.BlockSpec((_R, _D), lambda i: (i, 0)),
            pl.BlockSpec((_R, _D), lambda i: (i, 0)),
            pl.BlockSpec((1, _D), lambda i: (0, 0)),
            pl.BlockSpec((1, 1, _R), lambda i: (i, 0, 0)),
        ],
        out_specs=pl.BlockSpec((_G, _D), lambda i: (0, 0)),
        out_shape=jax.ShapeDtypeStruct((_G, _D), _f32),
        scratch_shapes=[
            pltpu.VMEM((_G, _D), _f32),
            pltpu.VMEM((_G, _D), _f32),
        ],
    )(acc, h4, dinv, b2.reshape(1, _D), batch3)


def kernel(x, edge_index, batch, W1, b1, Wm1, bm1, Wm2, bm2, W2, b2):
    pad = _EPAD - _E
    src3 = jnp.concatenate(
        [edge_index[0], jnp.zeros((pad,), jnp.int32)]).reshape(_NW, _CPW, _CHUNK)
    dst3 = jnp.concatenate(
        [edge_index[1], jnp.full((pad,), _N, jnp.int32)]).reshape(_NW, _CPW, _CHUNK)
    batch3 = batch.reshape(_N // _R, 1, _R)

    dacc = _sc_degree(dst3)
    dinv, h1 = _tc_prep(dacc, x, W1)
    acc1 = _sc_scatter(h1, src3, dst3)
    h2 = _tc_mid(acc1, h1, dinv, b1, Wm1)
    acc2 = _sc_scatter(h2, src3, dst3)
    h3 = _tc_mid(acc2, h2, dinv, bm1, Wm2)
    acc3 = _sc_scatter(h3, src3, dst3)
    h4 = _tc_mid(acc3, h3, dinv, bm2, W2)
    acc4 = _sc_scatter(h4, src3, dst3)
    return _tc_final(acc4, h4, dinv, b2, batch3)


# speedup vs baseline: 17.1623x; 2.5242x over previous
import functools

import jax
import jax.numpy as jnp
from jax import lax
from jax.experimental import pallas as pl
from jax.experimental.pallas import tpu as pltpu
from jax.experimental.pallas import tpu_sc as plsc

_N = 10000
_E = 320000
_D = 128
_G = 64

_NC = 2
_NS = 16
_NW = _NC * _NS
_CHUNK = 128

_CPW_DEG = 80
_EPAD_DEG = _NW * _CPW_DEG * _CHUNK

_CPW = 80
_EPAD = _NC * _NS * _CPW * _CHUNK

_ACC_ROWS = 10240
_ZROWS = _ACC_ROWS // _NS

_R = 1000

_f32 = jnp.float32
_mesh = functools.partial(
    plsc.VectorSubcoreMesh, core_axis_name="c", subcore_axis_name="s")


def _sc_degree_body(dst_hbm, out_hbm, dst_v, ones_v, zb_v, acc):
    c = lax.axis_index("c")
    s = lax.axis_index("s")
    w = c * _NS + s
    pltpu.sync_copy(dst_hbm.at[w], dst_v)

    @pl.loop(0, _CHUNK)
    def _fill(r):
        ones_v[r, :] = jnp.full((16,), 1.0, _f32)
        zb_v[r, :] = jnp.zeros((16,), _f32)

    @pl.loop(0, _ZROWS // _CHUNK)
    def _zero(k):
        pltpu.sync_copy(zb_v, acc.at[pl.ds(s * _ZROWS + k * _CHUNK, _CHUNK)])

    plsc.subcore_barrier()

    @pl.loop(0, _CPW_DEG)
    def _scatter(j):
        pltpu.sync_copy(ones_v, acc.at[dst_v.at[j]], add=True)

    plsc.subcore_barrier()
    pltpu.sync_copy(acc.at[pl.ds(s * _ZROWS, _ZROWS)],
                    out_hbm.at[c, pl.ds(s * _ZROWS, _ZROWS)])


def _sc_degree(dst3):
    return pl.kernel(
        _sc_degree_body,
        out_type=jax.ShapeDtypeStruct((_NC, _ACC_ROWS, 16), _f32),
        mesh=_mesh(),
        scratch_types=[
            pltpu.VMEM((_CPW_DEG, _CHUNK), jnp.int32),
            pltpu.VMEM((_CHUNK, 16), _f32),
            pltpu.VMEM((_CHUNK, 16), _f32),
            pltpu.VMEM_SHARED((_ACC_ROWS, 16), _f32),
        ],
    )(dst3)


def _sc_scatter_body(h_hbm, src_hbm, dst_hbm, out_hbm,
                     src_v, dst_v, rows_v, acc):
    c = lax.axis_index("c")
    s = lax.axis_index("s")

    @pl.loop(0, _CHUNK)
    def _fill(r):
        for cc in range(_D // 16):
            rows_v[r, pl.ds(cc * 16, 16)] = jnp.zeros((16,), _f32)

    @pl.loop(0, _ZROWS // _CHUNK)
    def _zero(k):
        pltpu.sync_copy(rows_v, acc.at[pl.ds(s * _ZROWS + k * _CHUNK, _CHUNK)])

    plsc.subcore_barrier()

    pltpu.sync_copy(src_hbm.at[c, s], src_v)
    pltpu.sync_copy(dst_hbm.at[c, s], dst_v)

    @pl.loop(0, _CPW)
    def _edges(j):
        pltpu.sync_copy(h_hbm.at[src_v.at[j]], rows_v)
        pltpu.sync_copy(rows_v, acc.at[dst_v.at[j]], add=True)

    plsc.subcore_barrier()
    pltpu.sync_copy(acc.at[pl.ds(s * _ZROWS, _ZROWS)],
                    out_hbm.at[c, pl.ds(s * _ZROWS, _ZROWS)])


def _sc_scatter(h, src4, dst4):
    return pl.kernel(
        _sc_scatter_body,
        out_type=jax.ShapeDtypeStruct((_NC, _ACC_ROWS, _D), _f32),
        mesh=_mesh(),
        scratch_types=[
            pltpu.VMEM((_CPW, _CHUNK), jnp.int32),
            pltpu.VMEM((_CPW, _CHUNK), jnp.int32),
            pltpu.VMEM((_CHUNK, _D), _f32),
            pltpu.VMEM_SHARED((_ACC_ROWS, _D), _f32),
        ],
    )(h, src4, dst4)


def _dot(a, b):
    return lax.dot_general(a, b, (((1,), (0,)), ((), ())),
                           preferred_element_type=_f32,
                           precision=lax.Precision.HIGHEST)


def _tc_prep_body(dacc_ref, x_ref, w_ref, dinv_ref, h_ref):
    deg = dacc_ref[0, :, 0:1] + dacc_ref[1, :, 0:1] + 1.0
    dinv = jnp.broadcast_to(lax.rsqrt(deg), (_R, _D))
    dinv_ref[...] = dinv
    h_ref[...] = _dot(x_ref[...], w_ref[...]) * dinv


def _tc_prep(dacc, x, w1):
    return pl.pallas_call(
        _tc_prep_body,
        grid=(_N // _R,),
        in_specs=[
            pl.BlockSpec((_NC, _R, 16), lambda i: (0, i, 0)),
            pl.BlockSpec((_R, _D), lambda i: (i, 0)),
            pl.BlockSpec((_D, _D), lambda i: (0, 0)),
        ],
        out_specs=[
            pl.BlockSpec((_R, _D), lambda i: (i, 0)),
            pl.BlockSpec((_R, _D), lambda i: (i, 0)),
        ],
        out_shape=[
            jax.ShapeDtypeStruct((_N, _D), _f32),
            jax.ShapeDtypeStruct((_N, _D), _f32),
        ],
    )(dacc, x, w1)


def _tc_mid_body(acc_ref, hprev_ref, dinv_ref, b_ref, w_ref, h_ref):
    d = dinv_ref[...]
    xr = jnp.maximum(
        d * (acc_ref[0] + acc_ref[1] + hprev_ref[...]) + b_ref[...], 0.0)
    h_ref[...] = _dot(xr, w_ref[...]) * d


def _tc_mid(acc, hprev, dinv, b, w):
    return pl.pallas_call(
        _tc_mid_body,
        grid=(_N // _R,),
        in_specs=[
            pl.BlockSpec((_NC, _R, _D), lambda i: (0, i, 0)),
            pl.BlockSpec((_R, _D), lambda i: (i, 0)),
            pl.BlockSpec((_R, _D), lambda i: (i, 0)),
            pl.BlockSpec((1, _D), lambda i: (0, 0)),
            pl.BlockSpec((_D, _D), lambda i: (0, 0)),
        ],
        out_specs=pl.BlockSpec((_R, _D), lambda i: (i, 0)),
        out_shape=jax.ShapeDtypeStruct((_N, _D), _f32),
    )(acc, hprev, dinv, b.reshape(1, _D), w)


def _tc_final_body(acc_ref, h4_ref, dinv_ref, b_ref,
                   batch_ref, out_ref, sum_acc, cnt_acc):
    i = pl.program_id(0)
    d = dinv_ref[...]
    hf = d * (acc_ref[0] + acc_ref[1] + h4_ref[...]) + b_ref[...]
    bt = batch_ref[0, 0, :]
    gids = lax.broadcasted_iota(jnp.int32, (_G, _R), 0)
    onehot = (gids == bt[None, :]).astype(_f32)
    ps = _dot(onehot, hf)
    cnt = jnp.broadcast_to(jnp.sum(onehot, axis=1, keepdims=True), (_G, _D))

    @pl.when(i == 0)
    def _init():
        sum_acc[...] = ps
        cnt_acc[...] = cnt

    @pl.when(i > 0)
    def _accum():
        sum_acc[...] += ps
        cnt_acc[...] += cnt

    @pl.when(i == _N // _R - 1)
    def _finalize():
        pool = sum_acc[...] / jnp.maximum(cnt_acc[...], 1.0)
        nsq = jnp.sum(pool * pool, axis=1, keepdims=True)
        inv = 1.0 / jnp.maximum(jnp.sqrt(nsq), 1e-12)
        out_ref[...] = pool * inv


def _tc_final(acc, h4, dinv, b2, batch3):
    return pl.pallas_call(
        _tc_final_body,
        grid=(_N // _R,),
        in_specs=[
            pl.BlockSpec((_NC, _R, _D), lambda i: (0, i, 0)),
            pl.BlockSpec((_R, _D), lambda i: (i, 0)),
            pl.BlockSpec((_R, _D), lambda i: (i, 0)),
            pl.BlockSpec((1, _D), lambda i: (0, 0)),
            pl.BlockSpec((1, 1, _R), lambda i: (i, 0, 0)),
        ],
        out_specs=pl.BlockSpec((_G, _D), lambda i: (0, 0)),
        out_shape=jax.ShapeDtypeStruct((_G, _D), _f32),
        scratch_shapes=[
            pltpu.VMEM((_G, _D), _f32),
            pltpu.VMEM((_G, _D), _f32),
        ],
    )(acc, h4, dinv, b2.reshape(1, _D), batch3)


def kernel(x, edge_index, batch, W1, b1, Wm1, bm1, Wm2, bm2, W2, b2):
    pad_d = _EPAD_DEG - _E
    dst3 = jnp.concatenate(
        [edge_index[1], _N + (jnp.arange(pad_d, dtype=jnp.int32) % (_ACC_ROWS - _N))]
    ).reshape(_NW, _CPW_DEG, _CHUNK)

    pad_e = _EPAD - _E
    pad_idx = jnp.arange(pad_e, dtype=jnp.int32)
    src4 = jnp.concatenate(
        [edge_index[0], pad_idx % _N]).reshape(_NC, _NS, _CPW, _CHUNK)
    dst4 = jnp.concatenate(
        [edge_index[1], _N + pad_idx % (_ACC_ROWS - _N)]).reshape(_NC, _NS, _CPW, _CHUNK)
    batch3 = batch.reshape(_N // _R, 1, _R)

    dacc = _sc_degree(dst3)
    dinv, h1 = _tc_prep(dacc, x, W1)
    acc1 = _sc_scatter(h1, src4, dst4)
    h2 = _tc_mid(acc1, h1, dinv, b1, Wm1)
    acc2 = _sc_scatter(h2, src4, dst4)
    h3 = _tc_mid(acc2, h2, dinv, bm1, Wm2)
    acc3 = _sc_scatter(h3, src4, dst4)
    h4 = _tc_mid(acc3, h3, dinv, bm2, W2)
    acc4 = _sc_scatter(h4, src4, dst4)
    return _tc_final(acc4, h4, dinv, b2, batch3)


# speedup vs baseline: 21.9296x; 1.2778x over previous
import functools

import jax
import jax.numpy as jnp
from jax import lax
from jax.experimental import pallas as pl
from jax.experimental.pallas import tpu as pltpu
from jax.experimental.pallas import tpu_sc as plsc

_N = 10000
_E = 320000
_D = 128
_G = 64

_NC = 2
_NS = 16
_NW = _NC * _NS
_CHUNK = 128

_CPW_DEG = 80
_EPAD_DEG = _NW * _CPW_DEG * _CHUNK

_CPW = 80
_CPH = _CPW // 2
_EPAD = _NC * _NS * _CPW * _CHUNK

_ACC_ROWS = 10240
_ZROWS = _ACC_ROWS // _NS

_R = 1000

_f32 = jnp.float32
_mesh = functools.partial(
    plsc.VectorSubcoreMesh, core_axis_name="c", subcore_axis_name="s")


def _sc_degree_body(dst_hbm, out_hbm, dst_v, ones_v, zb_v, acc):
    c = lax.axis_index("c")
    s = lax.axis_index("s")
    w = c * _NS + s
    pltpu.sync_copy(dst_hbm.at[w], dst_v)

    @pl.loop(0, _CHUNK)
    def _fill(r):
        ones_v[r, :] = jnp.full((16,), 1.0, _f32)
        zb_v[r, :] = jnp.zeros((16,), _f32)

    @pl.loop(0, _ZROWS // _CHUNK)
    def _zero(k):
        pltpu.sync_copy(zb_v, acc.at[pl.ds(s * _ZROWS + k * _CHUNK, _CHUNK)])

    plsc.subcore_barrier()

    @pl.loop(0, _CPW_DEG)
    def _scatter(j):
        pltpu.sync_copy(ones_v, acc.at[dst_v.at[j]], add=True)

    plsc.subcore_barrier()
    pltpu.sync_copy(acc.at[pl.ds(s * _ZROWS, _ZROWS)],
                    out_hbm.at[c, pl.ds(s * _ZROWS, _ZROWS)])


def _sc_degree(dst3):
    return pl.kernel(
        _sc_degree_body,
        out_type=jax.ShapeDtypeStruct((_NC, _ACC_ROWS, 16), _f32),
        mesh=_mesh(),
        scratch_types=[
            pltpu.VMEM((_CPW_DEG, _CHUNK), jnp.int32),
            pltpu.VMEM((_CHUNK, 16), _f32),
            pltpu.VMEM((_CHUNK, 16), _f32),
            pltpu.VMEM_SHARED((_ACC_ROWS, 16), _f32),
        ],
    )(dst3)


def _sc_scatter_body(h_hbm, src_hbm, dst_hbm, out_hbm,
                     src_v, dst_v, rows0, rows1, sem0, sem1, acc):
    c = lax.axis_index("c")
    s = lax.axis_index("s")

    @pl.loop(0, _CHUNK)
    def _fill(r):
        for cc in range(_D // 16):
            rows0[r, pl.ds(cc * 16, 16)] = jnp.zeros((16,), _f32)

    @pl.loop(0, _ZROWS // _CHUNK)
    def _zero(k):
        pltpu.sync_copy(rows0, acc.at[pl.ds(s * _ZROWS + k * _CHUNK, _CHUNK)])

    plsc.subcore_barrier()

    pltpu.sync_copy(dst_hbm.at[c, s], dst_v)

    for h in range(2):
        pltpu.sync_copy(src_hbm.at[c, s, h], src_v)

        @pl.loop(0, _CPH // 2)
        def _edges(i):
            for b, rows, sem in ((0, rows0, sem0), (1, rows1, sem1)):
                jl = i * 2 + b
                if h == 0:
                    @pl.when(i > 0)
                    def _drain():
                        pltpu.make_async_copy(
                            h_hbm.at[src_v.at[0]], rows, sem).wait()
                else:
                    pltpu.make_async_copy(
                        h_hbm.at[src_v.at[0]], rows, sem).wait()
                pltpu.sync_copy(h_hbm.at[src_v.at[jl]], rows)
                pltpu.async_copy(
                    rows, acc.at[dst_v.at[h * _CPH + jl]], sem, add=True)

    pltpu.make_async_copy(h_hbm.at[src_v.at[0]], rows0, sem0).wait()
    pltpu.make_async_copy(h_hbm.at[src_v.at[0]], rows1, sem1).wait()
    plsc.subcore_barrier()
    pltpu.sync_copy(acc.at[pl.ds(s * _ZROWS, _ZROWS)],
                    out_hbm.at[c, pl.ds(s * _ZROWS, _ZROWS)])


def _sc_scatter(h, src4, dst4):
    return pl.kernel(
        _sc_scatter_body,
        out_type=jax.ShapeDtypeStruct((_NC, _ACC_ROWS, _D), _f32),
        mesh=_mesh(),
        scratch_types=[
            pltpu.VMEM((_CPH, _CHUNK), jnp.int32),
            pltpu.VMEM((_CPW, _CHUNK), jnp.int32),
            pltpu.VMEM((_CHUNK, _D), _f32),
            pltpu.VMEM((_CHUNK, _D), _f32),
            pltpu.SemaphoreType.DMA,
            pltpu.SemaphoreType.DMA,
            pltpu.VMEM_SHARED((_ACC_ROWS, _D), _f32),
        ],
    )(h, src4, dst4)


def _dot(a, b):
    return lax.dot_general(a, b, (((1,), (0,)), ((), ())),
                           preferred_element_type=_f32,
                           precision=lax.Precision.HIGHEST)


def _tc_prep_body(dacc_ref, x_ref, w_ref, dinv_ref, h_ref):
    deg = dacc_ref[0, :, 0:1] + dacc_ref[1, :, 0:1] + 1.0
    dinv = jnp.broadcast_to(lax.rsqrt(deg), (_R, _D))
    dinv_ref[...] = dinv
    h_ref[...] = _dot(x_ref[...], w_ref[...]) * dinv


def _tc_prep(dacc, x, w1):
    return pl.pallas_call(
        _tc_prep_body,
        grid=(_N // _R,),
        in_specs=[
            pl.BlockSpec((_NC, _R, 16), lambda i: (0, i, 0)),
            pl.BlockSpec((_R, _D), lambda i: (i, 0)),
            pl.BlockSpec((_D, _D), lambda i: (0, 0)),
        ],
        out_specs=[
            pl.BlockSpec((_R, _D), lambda i: (i, 0)),
            pl.BlockSpec((_R, _D), lambda i: (i, 0)),
        ],
        out_shape=[
            jax.ShapeDtypeStruct((_N, _D), _f32),
            jax.ShapeDtypeStruct((_N, _D), _f32),
        ],
    )(dacc, x, w1)


def _tc_mid_body(acc_ref, hprev_ref, dinv_ref, b_ref, w_ref, h_ref):
    d = dinv_ref[...]
    xr = jnp.maximum(
        d * (acc_ref[0] + acc_ref[1] + hprev_ref[...]) + b_ref[...], 0.0)
    h_ref[...] = _dot(xr, w_ref[...]) * d


def _tc_mid(acc, hprev, dinv, b, w):
    return pl.pallas_call(
        _tc_mid_body,
        grid=(_N // _R,),
        in_specs=[
            pl.BlockSpec((_NC, _R, _D), lambda i: (0, i, 0)),
            pl.BlockSpec((_R, _D), lambda i: (i, 0)),
            pl.BlockSpec((_R, _D), lambda i: (i, 0)),
            pl.BlockSpec((1, _D), lambda i: (0, 0)),
            pl.BlockSpec((_D, _D), lambda i: (0, 0)),
        ],
        out_specs=pl.BlockSpec((_R, _D), lambda i: (i, 0)),
        out_shape=jax.ShapeDtypeStruct((_N, _D), _f32),
    )(acc, hprev, dinv, b.reshape(1, _D), w)


def _tc_final_body(acc_ref, h4_ref, dinv_ref, b_ref,
                   batch_ref, out_ref, sum_acc, cnt_acc):
    i = pl.program_id(0)
    d = dinv_ref[...]
    hf = d * (acc_ref[0] + acc_ref[1] + h4_ref[...]) + b_ref[...]
    bt = batch_ref[0, 0, :]
    gids = lax.broadcasted_iota(jnp.int32, (_G, _R), 0)
    onehot = (gids == bt[None, :]).astype(_f32)
    ps = _dot(onehot, hf)
    cnt = jnp.broadcast_to(jnp.sum(onehot, axis=1, keepdims=True), (_G, _D))

    @pl.when(i == 0)
    def _init():
        sum_acc[...] = ps
        cnt_acc[...] = cnt

    @pl.when(i > 0)
    def _accum():
        sum_acc[...] += ps
        cnt_acc[...] += cnt

    @pl.when(i == _N // _R - 1)
    def _finalize():
        pool = sum_acc[...] / jnp.maximum(cnt_acc[...], 1.0)
        nsq = jnp.sum(pool * pool, axis=1, keepdims=True)
        inv = 1.0 / jnp.maximum(jnp.sqrt(nsq), 1e-12)
        out_ref[...] = pool * inv


def _tc_final(acc, h4, dinv, b2, batch3):
    return pl.pallas_call(
        _tc_final_body,
        grid=(_N // _R,),
        in_specs=[
            pl.BlockSpec((_NC, _R, _D), lambda i: (0, i, 0)),
            pl.BlockSpec((_R, _D), lambda i: (i, 0)),
            pl.BlockSpec((_R, _D), lambda i: (i, 0)),
            pl.BlockSpec((1, _D), lambda i: (0, 0)),
            pl.BlockSpec((1, 1, _R), lambda i: (i, 0, 0)),
        ],
        out_specs=pl.BlockSpec((_G, _D), lambda i: (0, 0)),
        out_shape=jax.ShapeDtypeStruct((_G, _D), _f32),
        scratch_shapes=[
            pltpu.VMEM((_G, _D), _f32),
            pltpu.VMEM((_G, _D), _f32),
        ],
    )(acc, h4, dinv, b2.reshape(1, _D), batch3)


def kernel(x, edge_index, batch, W1, b1, Wm1, bm1, Wm2, bm2, W2, b2):
    pad_d = _EPAD_DEG - _E
    dst3 = jnp.concatenate(
        [edge_index[1], _N + (jnp.arange(pad_d, dtype=jnp.int32) % (_ACC_ROWS - _N))]
    ).reshape(_NW, _CPW_DEG, _CHUNK)

    pad_e = _EPAD - _E
    pad_idx = jnp.arange(pad_e, dtype=jnp.int32)
    src4 = jnp.concatenate(
        [edge_index[0], pad_idx % _N]).reshape(_NC, _NS, 2, _CPH, _CHUNK)
    dst4 = jnp.concatenate(
        [edge_index[1], _N + pad_idx % (_ACC_ROWS - _N)]).reshape(_NC, _NS, _CPW, _CHUNK)
    batch3 = batch.reshape(_N // _R, 1, _R)

    dacc = _sc_degree(dst3)
    dinv, h1 = _tc_prep(dacc, x, W1)
    acc1 = _sc_scatter(h1, src4, dst4)
    h2 = _tc_mid(acc1, h1, dinv, b1, Wm1)
    acc2 = _sc_scatter(h2, src4, dst4)
    h3 = _tc_mid(acc2, h2, dinv, bm1, Wm2)
    acc3 = _sc_scatter(h3, src4, dst4)
    h4 = _tc_mid(acc3, h3, dinv, bm2, W2)
    acc4 = _sc_scatter(h4, src4, dst4)
    return _tc_final(acc4, h4, dinv, b2, batch3)
